# baseline (speedup 1.0000x reference)
import functools

import jax
import jax.numpy as jnp
from jax import lax
from jax.experimental import pallas as pl
from jax.experimental.pallas import tpu as pltpu
from jax.experimental.pallas import tpu_sc as plsc

N = 10000
D = 128
NC = 2
NS = 16
CHUNK = 128
LANES = 16
NB = 3
NE = 4
_UNROLL = 12


_BLK = 1000


def _mlp_body(h, wbs):
    for w, b in wbs:
        h = jnp.maximum(jnp.dot(h, w[...], preferred_element_type=jnp.float32)
                        + b[...], 0.0)
    return h


def _mlp3_tc(x, W0, b0, W1, b1, W2, b2):
    grid = (N // _BLK,)
    wspec = pl.BlockSpec((D, D), lambda i: (0, 0))
    bspec = pl.BlockSpec((1, D), lambda i: (0, 0))

    def body(x_ref, w0, b0r, w1, b1r, w2, b2r, o_ref):
        o_ref[...] = _mlp_body(x_ref[...], ((w0, b0r), (w1, b1r), (w2, b2r)))

    return pl.pallas_call(
        body,
        grid=grid,
        in_specs=[pl.BlockSpec((_BLK, D), lambda i: (i, 0)),
                  wspec, bspec, wspec, bspec, wspec, bspec],
        out_specs=pl.BlockSpec((_BLK, D), lambda i: (i, 0)),
        out_shape=jax.ShapeDtypeStruct((N, D), jnp.float32),
    )(x, W0, b0.reshape(1, D), W1, b1.reshape(1, D), W2, b2.reshape(1, D))


def _combine_mlp3_tc(parts, W0, b0, W1, b1, W2, b2):
    grid = (N // _BLK,)
    wspec = pl.BlockSpec((D, D), lambda i: (0, 0))
    bspec = pl.BlockSpec((1, D), lambda i: (0, 0))

    def body(p0_ref, p1_ref, w0, b0r, w1, b1r, w2, b2r, s_ref, h_ref):
        x = p0_ref[0] + p1_ref[0]
        s_ref[...] = x
        h_ref[...] = _mlp_body(x, ((w0, b0r), (w1, b1r), (w2, b2r)))

    return pl.pallas_call(
        body,
        grid=grid,
        in_specs=[pl.BlockSpec((1, _BLK, D), lambda i: (0, i, 0)),
                  pl.BlockSpec((1, _BLK, D), lambda i: (1, i, 0)),
                  wspec, bspec, wspec, bspec, wspec, bspec],
        out_specs=[pl.BlockSpec((_BLK, D), lambda i: (i, 0)),
                   pl.BlockSpec((_BLK, D), lambda i: (i, 0))],
        out_shape=[jax.ShapeDtypeStruct((N, D), jnp.float32),
                   jax.ShapeDtypeStruct((N, D), jnp.float32)],
    )(parts, parts, W0, b0.reshape(1, D), W1, b1.reshape(1, D),
      W2, b2.reshape(1, D))


def _combine_tc(parts):
    grid = (N // _BLK,)

    def body(p0_ref, p1_ref, o_ref):
        o_ref[...] = p0_ref[0] + p1_ref[0]

    return pl.pallas_call(
        body,
        grid=grid,
        in_specs=[pl.BlockSpec((1, _BLK, D), lambda i: (0, i, 0)),
                  pl.BlockSpec((1, _BLK, D), lambda i: (1, i, 0))],
        out_specs=pl.BlockSpec((_BLK, D), lambda i: (i, 0)),
        out_shape=jax.ShapeDtypeStruct((N, D), jnp.float32),
    )(parts, parts)


def _prep_edges(indices, values):
    e = values.shape[0]
    per_tile = -(-e // (NC * NS * CHUNK * _UNROLL)) * CHUNK * _UNROLL
    pad = per_tile * NC * NS - e
    row = jnp.concatenate([indices[0], jnp.zeros((pad,), jnp.int32)])
    col = jnp.concatenate([indices[1], jnp.zeros((pad,), jnp.int32)])
    val = jnp.concatenate([values, jnp.zeros((pad,), jnp.float32)])
    shape = (NC, NS, per_tile // CHUNK, 1, CHUNK)
    cr = jnp.concatenate([col.reshape(shape), row.reshape(shape)], axis=3)
    return cr, val.reshape(shape[:3] + (CHUNK,))


def _spmm_sc(h, packed, vals):
    nchunks = packed.shape[2]
    rpt = (N // NS) // 8 * 8
    tail = N - NS * rpt
    nz_full = rpt // CHUNK
    nz_rem = rpt % CHUNK
    mesh = plsc.VectorSubcoreMesh(core_axis_name="c", subcore_axis_name="s",
                                  num_subcores=NS)

    @functools.partial(
        pl.kernel,
        out_type=jax.ShapeDtypeStruct((NC, N, D), jnp.float32),
        mesh=mesh,
        scratch_types=[
            pltpu.VMEM((NE, 2, CHUNK), jnp.int32),
            pltpu.VMEM((NE, CHUNK), jnp.float32),
            pltpu.VMEM((NB, CHUNK, D), jnp.float32),
            pltpu.VMEM_SHARED((N, D), jnp.float32),
            [pltpu.SemaphoreType.DMA] * NE,
            [pltpu.SemaphoreType.DMA] * NB,
            [pltpu.SemaphoreType.DMA] * NB,
        ],
    )
    def k(h_hbm, p_hbm, v_hbm, out_hbm, ebuf, vbuf, gbuf, acc,
          esems, gsems, ssems):
        c = lax.axis_index("c")
        s = lax.axis_index("s")

        def issue_meta(j, ne):
            pltpu.async_copy(p_hbm.at[c, s, j], ebuf.at[ne], esems[ne])
            pltpu.async_copy(v_hbm.at[c, s, j], vbuf.at[ne], esems[ne])

        def wait_meta(j, ne):
            pltpu.make_async_copy(p_hbm.at[c, s, j], ebuf.at[ne],
                                  esems[ne]).wait()
            pltpu.make_async_copy(v_hbm.at[c, s, j], vbuf.at[ne],
                                  esems[ne]).wait()

        def issue_gather(b, ne):
            pltpu.async_copy(h_hbm.at[ebuf.at[ne, 0]], gbuf.at[b], gsems[b])

        def wait_gather(b, ne):
            pltpu.make_async_copy(h_hbm.at[ebuf.at[ne, 0]], gbuf.at[b],
                                  gsems[b]).wait()

        def issue_scatter(b, ne):
            pltpu.async_copy(gbuf.at[b], acc.at[ebuf.at[ne, 1]], ssems[b],
                             add=True)

        def wait_scatter(b, ne):
            pltpu.make_async_copy(gbuf.at[b], acc.at[ebuf.at[ne, 1]],
                                  ssems[b]).wait()

        for j in range(NE):
            issue_meta(j, j)

        zero = jnp.zeros((LANES,), jnp.float32)

        def zbody(r, _):
            for q in range(D // LANES):
                gbuf[0, r, pl.ds(q * LANES, LANES)] = zero
            return 0

        lax.fori_loop(0, CHUNK, zbody, 0)
        base = s * rpt
        for t in range(nz_full):
            pltpu.sync_copy(gbuf.at[0], acc.at[pl.ds(base + t * CHUNK, CHUNK)])
        if nz_rem:
            pltpu.sync_copy(gbuf.at[0, pl.ds(0, nz_rem)],
                            acc.at[pl.ds(base + nz_full * CHUNK, nz_rem)])
        if tail:
            @pl.when(s == NS - 1)
            def _():
                pltpu.sync_copy(gbuf.at[0, pl.ds(0, tail)],
                                acc.at[pl.ds(NS * rpt, tail)])

        for j in range(NB):
            wait_meta(j, j)
            issue_gather(j, j)
        plsc.subcore_barrier()

        dnums = lax.GatherDimensionNumbers(
            offset_dims=(), collapsed_slice_dims=(0,), start_index_map=(0,))

        def scale(b, ne):
            def ebody(e, _):
                g16 = jnp.bitwise_and(e, -LANES)
                lane = jnp.bitwise_and(e, LANES - 1)
                vv = vbuf[ne, pl.ds(g16, LANES)]
                bc = lax.gather(
                    vv, jnp.full((LANES, 1), 0, jnp.int32) + lane, dnums,
                    slice_sizes=(1,),
                    mode=lax.GatherScatterMode.PROMISE_IN_BOUNDS)
                for q in range(D // LANES):
                    sl = pl.ds(q * LANES, LANES)
                    gbuf[b, e, sl] = gbuf[b, e, sl] * bc
                return 0

            lax.fori_loop(0, CHUNK, ebody, 0)

        def obody(o, _):
            for u in range(_UNROLL):
                j = o * _UNROLL + u
                b = u % NB
                ne = u % NE
                wait_gather(b, ne)
                scale(b, ne)
                issue_scatter(b, ne)
                bp = (u - 1) % NB
                nep = (u - 1) % NE
                nejn = (u + NB - 1) % NE
                jn = j + NB - 1
                jf = j + NE - 1
                nef = (u + NE - 1) % NE

                def tail_steps(guard_jn, guard_jf):
                    def _go():
                        wait_scatter(bp, nep)

                        @pl.when(guard_jn)
                        def _():
                            wait_meta(jn, nejn)
                            issue_gather(bp, nejn)

                        @pl.when(guard_jf)
                        def _():
                            issue_meta(jf, nef)

                    return _go

                if u == 0:
                    pl.when(j > 0)(tail_steps(jn < nchunks, jf < nchunks))
                else:
                    tail_steps(jn < nchunks, jf < nchunks)()
            return 0

        lax.fori_loop(0, nchunks // _UNROLL, obody, 0)
        wait_scatter((nchunks - 1) % NB, (nchunks - 1) % NE)
        plsc.subcore_barrier()

        pltpu.sync_copy(acc.at[pl.ds(base, rpt)],
                        out_hbm.at[c, pl.ds(base, rpt)])
        if tail:
            @pl.when(s == NS - 1)
            def _():
                pltpu.sync_copy(acc.at[pl.ds(NS * rpt, tail)],
                                out_hbm.at[c, pl.ds(NS * rpt, tail)])

    return k(h, packed, vals)


def kernel(inputs, summ0_indices, summ0_values, summ1_indices, summ1_values,
           dag_W0, dag_b0, dag_W1, dag_b1, dag_W2, dag_b2,
           glob_W0, glob_b0, glob_W1, glob_b1, glob_W2, glob_b2):
    h0 = _mlp3_tc(inputs, dag_W0, dag_b0, dag_W1, dag_b1, dag_W2, dag_b2)
    p0, v0 = _prep_edges(summ0_indices, summ0_values)
    parts0 = _spmm_sc(h0, p0, v0)
    out0, h1 = _combine_mlp3_tc(parts0, glob_W0, glob_b0, glob_W1, glob_b1,
                                glob_W2, glob_b2)
    p1, v1 = _prep_edges(summ1_indices, summ1_values)
    parts1 = _spmm_sc(h1, p1, v1)
    out1 = _combine_tc(parts1)
    return (out0, out1)


# speedup vs baseline: 2.4809x; 2.4809x over previous
import functools

import jax
import jax.numpy as jnp
from jax import lax
from jax.experimental import pallas as pl
from jax.experimental.pallas import tpu as pltpu
from jax.experimental.pallas import tpu_sc as plsc

N = 10000
D = 128
NC = 2
NS = 16
CHUNK = 128
LANES = 16
NB = 3
NE = 4


_BLK = 1000


def _mlp_body(h, wbs):
    for w, b in wbs:
        h = jnp.maximum(jnp.dot(h, w[...], preferred_element_type=jnp.float32)
                        + b[...], 0.0)
    return h


def _mlp3_tc(x, W0, b0, W1, b1, W2, b2):
    grid = (N // _BLK,)
    wspec = pl.BlockSpec((D, D), lambda i: (0, 0))
    bspec = pl.BlockSpec((1, D), lambda i: (0, 0))

    def body(x_ref, w0, b0r, w1, b1r, w2, b2r, o_ref):
        o_ref[...] = _mlp_body(x_ref[...], ((w0, b0r), (w1, b1r), (w2, b2r)))

    return pl.pallas_call(
        body,
        grid=grid,
        in_specs=[pl.BlockSpec((_BLK, D), lambda i: (i, 0)),
                  wspec, bspec, wspec, bspec, wspec, bspec],
        out_specs=pl.BlockSpec((_BLK, D), lambda i: (i, 0)),
        out_shape=jax.ShapeDtypeStruct((N, D), jnp.float32),
    )(x, W0, b0.reshape(1, D), W1, b1.reshape(1, D), W2, b2.reshape(1, D))


def _combine_mlp3_tc(parts, W0, b0, W1, b1, W2, b2):
    grid = (N // _BLK,)
    wspec = pl.BlockSpec((D, D), lambda i: (0, 0))
    bspec = pl.BlockSpec((1, D), lambda i: (0, 0))

    def body(p0_ref, p1_ref, w0, b0r, w1, b1r, w2, b2r, s_ref, h_ref):
        x = p0_ref[0] + p1_ref[0]
        s_ref[...] = x
        h_ref[...] = _mlp_body(x, ((w0, b0r), (w1, b1r), (w2, b2r)))

    return pl.pallas_call(
        body,
        grid=grid,
        in_specs=[pl.BlockSpec((1, _BLK, D), lambda i: (0, i, 0)),
                  pl.BlockSpec((1, _BLK, D), lambda i: (1, i, 0)),
                  wspec, bspec, wspec, bspec, wspec, bspec],
        out_specs=[pl.BlockSpec((_BLK, D), lambda i: (i, 0)),
                   pl.BlockSpec((_BLK, D), lambda i: (i, 0))],
        out_shape=[jax.ShapeDtypeStruct((N, D), jnp.float32),
                   jax.ShapeDtypeStruct((N, D), jnp.float32)],
    )(parts, parts, W0, b0.reshape(1, D), W1, b1.reshape(1, D),
      W2, b2.reshape(1, D))


def _combine_tc(parts):
    grid = (N // _BLK,)

    def body(p0_ref, p1_ref, o_ref):
        o_ref[...] = p0_ref[0] + p1_ref[0]

    return pl.pallas_call(
        body,
        grid=grid,
        in_specs=[pl.BlockSpec((1, _BLK, D), lambda i: (0, i, 0)),
                  pl.BlockSpec((1, _BLK, D), lambda i: (1, i, 0))],
        out_specs=pl.BlockSpec((_BLK, D), lambda i: (i, 0)),
        out_shape=jax.ShapeDtypeStruct((N, D), jnp.float32),
    )(parts, parts)


def _prep_edges(indices, values):
    e = values.shape[0]
    per_tile = -(-e // (NC * NS * CHUNK)) * CHUNK
    pad = per_tile * NC * NS - e
    row = jnp.concatenate([indices[0], jnp.zeros((pad,), jnp.int32)])
    col = jnp.concatenate([indices[1], jnp.zeros((pad,), jnp.int32)])
    val = jnp.concatenate([values, jnp.zeros((pad,), jnp.float32)])
    shape = (NC, NS, per_tile // CHUNK, 1, CHUNK)
    cr = jnp.concatenate([col.reshape(shape), row.reshape(shape)], axis=3)
    return cr, val.reshape(shape[:3] + (CHUNK,))


def _spmm_sc(h, packed, vals):
    nchunks = packed.shape[2]
    rpt = (N // NS) // 8 * 8
    tail = N - NS * rpt
    nz_full = rpt // CHUNK
    nz_rem = rpt % CHUNK
    mesh = plsc.VectorSubcoreMesh(core_axis_name="c", subcore_axis_name="s",
                                  num_subcores=NS)

    @functools.partial(
        pl.kernel,
        out_type=jax.ShapeDtypeStruct((NC, N, D), jnp.float32),
        mesh=mesh,
        scratch_types=[
            pltpu.VMEM((NE, 2, CHUNK), jnp.int32),
            pltpu.VMEM((NE, CHUNK), jnp.float32),
            pltpu.VMEM((NB, CHUNK, D), jnp.float32),
            pltpu.VMEM_SHARED((N, D), jnp.float32),
            pltpu.SemaphoreType.DMA((NE,)),
            pltpu.SemaphoreType.DMA((NB,)),
        ],
    )
    def k(h_hbm, p_hbm, v_hbm, out_hbm, ebuf, vbuf, gbuf, acc,
          esems, gsems):
        c = lax.axis_index("c")
        s = lax.axis_index("s")

        def issue_meta(j, ne):
            pltpu.async_copy(p_hbm.at[c, s, j], ebuf.at[ne], esems.at[ne])
            pltpu.async_copy(v_hbm.at[c, s, j], vbuf.at[ne], esems.at[ne])

        def wait_meta(j, ne):
            pltpu.make_async_copy(p_hbm.at[c, s, j], ebuf.at[ne],
                                  esems.at[ne]).wait()
            pltpu.make_async_copy(v_hbm.at[c, s, j], vbuf.at[ne],
                                  esems.at[ne]).wait()

        def issue_gather(b, ne):
            pltpu.async_copy(h_hbm.at[ebuf.at[ne, 0]], gbuf.at[b],
                             gsems.at[b])

        def wait_gather(b, ne):
            pltpu.make_async_copy(h_hbm.at[ebuf.at[ne, 0]], gbuf.at[b],
                                  gsems.at[b]).wait()

        def scatter(b, ne):
            pltpu.sync_copy(gbuf.at[b], acc.at[ebuf.at[ne, 1]], add=True)

        for j in range(NE):
            issue_meta(j, j)

        zero = jnp.zeros((LANES,), jnp.float32)

        def zbody(r, _):
            for q in range(D // LANES):
                gbuf[0, r, pl.ds(q * LANES, LANES)] = zero
            return 0

        lax.fori_loop(0, CHUNK, zbody, 0)
        base = s * rpt
        for t in range(nz_full):
            pltpu.sync_copy(gbuf.at[0], acc.at[pl.ds(base + t * CHUNK, CHUNK)])
        if nz_rem:
            pltpu.sync_copy(gbuf.at[0, pl.ds(0, nz_rem)],
                            acc.at[pl.ds(base + nz_full * CHUNK, nz_rem)])
        if tail:
            @pl.when(s == NS - 1)
            def _():
                pltpu.sync_copy(gbuf.at[0, pl.ds(0, tail)],
                                acc.at[pl.ds(NS * rpt, tail)])

        for j in range(NB):
            wait_meta(j, j)
            issue_gather(j, j)
        plsc.subcore_barrier()

        dnums = lax.GatherDimensionNumbers(
            offset_dims=(), collapsed_slice_dims=(0,), start_index_map=(0,))

        def scale(b, ne):
            def gbody(g, _):
                vv = vbuf[ne, pl.ds(g * LANES, LANES)]
                for i in range(LANES):
                    bc = lax.gather(
                        vv, jnp.full((LANES, 1), i, jnp.int32), dnums,
                        slice_sizes=(1,),
                        mode=lax.GatherScatterMode.PROMISE_IN_BOUNDS)
                    e = g * LANES + i
                    for q in range(D // LANES):
                        sl = pl.ds(q * LANES, LANES)
                        gbuf[b, e, sl] = gbuf[b, e, sl] * bc
                return 0

            lax.fori_loop(0, CHUNK // LANES, gbody, 0)

        def jbody(j, _):
            b = lax.rem(j, NB)
            ne = lax.rem(j, NE)
            wait_gather(b, ne)
            scale(b, ne)
            scatter(b, ne)
            bp = lax.rem(j + (NB - 1), NB)
            jn = j + NB - 1
            jf = j + NE - 1
            nejn = lax.rem(jn, NE)
            nef = lax.rem(jf, NE)

            @pl.when(j > 0)
            def _():
                @pl.when(jn < nchunks)
                def _():
                    wait_meta(jn, nejn)
                    issue_gather(bp, nejn)

                @pl.when(jf < nchunks)
                def _():
                    issue_meta(jf, nef)

            return 0

        lax.fori_loop(0, nchunks, jbody, 0)
        plsc.subcore_barrier()

        pltpu.sync_copy(acc.at[pl.ds(base, rpt)],
                        out_hbm.at[c, pl.ds(base, rpt)])
        if tail:
            @pl.when(s == NS - 1)
            def _():
                pltpu.sync_copy(acc.at[pl.ds(NS * rpt, tail)],
                                out_hbm.at[c, pl.ds(NS * rpt, tail)])

    return k(h, packed, vals)


def kernel(inputs, summ0_indices, summ0_values, summ1_indices, summ1_values,
           dag_W0, dag_b0, dag_W1, dag_b1, dag_W2, dag_b2,
           glob_W0, glob_b0, glob_W1, glob_b1, glob_W2, glob_b2):
    h0 = _mlp3_tc(inputs, dag_W0, dag_b0, dag_W1, dag_b1, dag_W2, dag_b2)
    p0, v0 = _prep_edges(summ0_indices, summ0_values)
    parts0 = _spmm_sc(h0, p0, v0)
    out0, h1 = _combine_mlp3_tc(parts0, glob_W0, glob_b0, glob_W1, glob_b1,
                                glob_W2, glob_b2)
    p1, v1 = _prep_edges(summ1_indices, summ1_values)
    parts1 = _spmm_sc(h1, p1, v1)
    out1 = _combine_tc(parts1)
    return (out0, out1)


# speedup vs baseline: 3.0282x; 1.2206x over previous
import functools

import jax
import jax.numpy as jnp
from jax import lax
from jax.experimental import pallas as pl
from jax.experimental.pallas import tpu as pltpu
from jax.experimental.pallas import tpu_sc as plsc

N = 10000
D = 128
NC = 2
NS = 16
CHUNK = 128
LANES = 16


_BLK = 1000


def _mlp_body(h, wbs):
    for w, b in wbs:
        h = jnp.maximum(jnp.dot(h, w[...], preferred_element_type=jnp.float32)
                        + b[...], 0.0)
    return h


def _mlp3_tc(x, W0, b0, W1, b1, W2, b2):
    grid = (N // _BLK,)
    wspec = pl.BlockSpec((D, D), lambda i: (0, 0))
    bspec = pl.BlockSpec((1, D), lambda i: (0, 0))

    def body(x_ref, w0, b0r, w1, b1r, w2, b2r, o_ref):
        o_ref[...] = _mlp_body(x_ref[...], ((w0, b0r), (w1, b1r), (w2, b2r)))

    return pl.pallas_call(
        body,
        grid=grid,
        in_specs=[pl.BlockSpec((_BLK, D), lambda i: (i, 0)),
                  wspec, bspec, wspec, bspec, wspec, bspec],
        out_specs=pl.BlockSpec((_BLK, D), lambda i: (i, 0)),
        out_shape=jax.ShapeDtypeStruct((N, D), jnp.float32),
    )(x, W0, b0.reshape(1, D), W1, b1.reshape(1, D), W2, b2.reshape(1, D))


def _combine_mlp3_tc(parts, W0, b0, W1, b1, W2, b2):
    grid = (N // _BLK,)
    wspec = pl.BlockSpec((D, D), lambda i: (0, 0))
    bspec = pl.BlockSpec((1, D), lambda i: (0, 0))

    def body(p0_ref, p1_ref, w0, b0r, w1, b1r, w2, b2r, s_ref, h_ref):
        x = p0_ref[0] + p1_ref[0]
        s_ref[...] = x
        h_ref[...] = _mlp_body(x, ((w0, b0r), (w1, b1r), (w2, b2r)))

    return pl.pallas_call(
        body,
        grid=grid,
        in_specs=[pl.BlockSpec((1, _BLK, D), lambda i: (0, i, 0)),
                  pl.BlockSpec((1, _BLK, D), lambda i: (1, i, 0)),
                  wspec, bspec, wspec, bspec, wspec, bspec],
        out_specs=[pl.BlockSpec((_BLK, D), lambda i: (i, 0)),
                   pl.BlockSpec((_BLK, D), lambda i: (i, 0))],
        out_shape=[jax.ShapeDtypeStruct((N, D), jnp.float32),
                   jax.ShapeDtypeStruct((N, D), jnp.float32)],
    )(parts, parts, W0, b0.reshape(1, D), W1, b1.reshape(1, D),
      W2, b2.reshape(1, D))


def _combine_tc(parts):
    grid = (N // _BLK,)

    def body(p0_ref, p1_ref, o_ref):
        o_ref[...] = p0_ref[0] + p1_ref[0]

    return pl.pallas_call(
        body,
        grid=grid,
        in_specs=[pl.BlockSpec((1, _BLK, D), lambda i: (0, i, 0)),
                  pl.BlockSpec((1, _BLK, D), lambda i: (1, i, 0))],
        out_specs=pl.BlockSpec((_BLK, D), lambda i: (i, 0)),
        out_shape=jax.ShapeDtypeStruct((N, D), jnp.float32),
    )(parts, parts)


def _prep_edges(indices, values):
    e = values.shape[0]
    per_tile = -(-e // (NC * NS * CHUNK)) * CHUNK
    epad = per_tile * NC * NS
    pad = epad - e
    row = jnp.concatenate([indices[0], jnp.zeros((pad,), jnp.int32)])
    col = jnp.concatenate([indices[1], jnp.zeros((pad,), jnp.int32)])
    val = jnp.concatenate([values, jnp.zeros((pad,), jnp.float32)])
    shape = (NC, NS, per_tile // CHUNK, CHUNK)
    return row.reshape(shape), col.reshape(shape), val.reshape(shape)


def _spmm_sc(h, row, col, val):
    nchunks = row.shape[2]
    rpt = (N // NS) // 8 * 8
    tail = N - NS * rpt
    nz_full = rpt // CHUNK
    nz_rem = rpt % CHUNK
    mesh = plsc.VectorSubcoreMesh(core_axis_name="c", subcore_axis_name="s")

    @functools.partial(
        pl.kernel,
        out_type=jax.ShapeDtypeStruct((NC, N, D), jnp.float32),
        mesh=mesh,
        scratch_types=[
            pltpu.VMEM((nchunks, CHUNK), jnp.int32),
            pltpu.VMEM((nchunks, CHUNK), jnp.int32),
            pltpu.VMEM((nchunks, CHUNK), jnp.float32),
            pltpu.VMEM((CHUNK, D), jnp.float32),
            pltpu.VMEM_SHARED((N, D), jnp.float32),
            pltpu.SemaphoreType.DMA,
        ],
    )
    def k(h_hbm, row_hbm, col_hbm, val_hbm, out_hbm,
          row_v, col_v, val_v, gbuf, acc, gsem):
        c = lax.axis_index("c")
        s = lax.axis_index("s")

        pltpu.sync_copy(row_hbm.at[c, s], row_v)
        pltpu.sync_copy(col_hbm.at[c, s], col_v)
        pltpu.sync_copy(val_hbm.at[c, s], val_v)

        zero = jnp.zeros((LANES,), jnp.float32)

        def zbody(r, _):
            for q in range(D // LANES):
                gbuf[r, pl.ds(q * LANES, LANES)] = zero
            return 0

        lax.fori_loop(0, CHUNK, zbody, 0)
        base = s * rpt
        for t in range(nz_full):
            pltpu.sync_copy(gbuf, acc.at[pl.ds(base + t * CHUNK, CHUNK)])
        if nz_rem:
            pltpu.sync_copy(gbuf.at[pl.ds(0, nz_rem)],
                            acc.at[pl.ds(base + nz_full * CHUNK, nz_rem)])
        if tail:
            @pl.when(s == NS - 1)
            def _():
                pltpu.sync_copy(gbuf.at[pl.ds(0, tail)],
                                acc.at[pl.ds(NS * rpt, tail)])
        plsc.subcore_barrier()

        def chunk_body(j, _):
            pltpu.async_copy(h_hbm.at[col_v.at[j]], gbuf, gsem).wait()

            dnums = lax.GatherDimensionNumbers(
                offset_dims=(), collapsed_slice_dims=(0,), start_index_map=(0,))

            def gbody(g, _):
                vv = val_v[j, pl.ds(g * LANES, LANES)]
                for i in range(LANES):
                    b = lax.gather(
                        vv, jnp.full((LANES, 1), i, jnp.int32), dnums,
                        slice_sizes=(1,),
                        mode=lax.GatherScatterMode.PROMISE_IN_BOUNDS)
                    e = g * LANES + i
                    for q in range(D // LANES):
                        sl = pl.ds(q * LANES, LANES)
                        gbuf[e, sl] = gbuf[e, sl] * b
                return 0

            lax.fori_loop(0, CHUNK // LANES, gbody, 0)
            pltpu.sync_copy(gbuf, acc.at[row_v.at[j]], add=True)
            return 0

        lax.fori_loop(0, nchunks, chunk_body, 0)
        plsc.subcore_barrier()

        pltpu.sync_copy(acc.at[pl.ds(base, rpt)],
                        out_hbm.at[c, pl.ds(base, rpt)])
        if tail:
            @pl.when(s == NS - 1)
            def _():
                pltpu.sync_copy(acc.at[pl.ds(NS * rpt, tail)],
                                out_hbm.at[c, pl.ds(NS * rpt, tail)])

    return k(h, row, col, val)


def kernel(inputs, summ0_indices, summ0_values, summ1_indices, summ1_values,
           dag_W0, dag_b0, dag_W1, dag_b1, dag_W2, dag_b2,
           glob_W0, glob_b0, glob_W1, glob_b1, glob_W2, glob_b2):
    h0 = _mlp3_tc(inputs, dag_W0, dag_b0, dag_W1, dag_b1, dag_W2, dag_b2)
    r0, c0, v0 = _prep_edges(summ0_indices, summ0_values)
    parts0 = _spmm_sc(h0, r0, c0, v0)
    out0, h1 = _combine_mlp3_tc(parts0, glob_W0, glob_b0, glob_W1, glob_b1,
                                glob_W2, glob_b2)
    r1, c1, v1 = _prep_edges(summ1_indices, summ1_values)
    parts1 = _spmm_sc(h1, r1, c1, v1)
    out1 = _combine_tc(parts1)
    return (out0, out1)


# speedup vs baseline: 3.0683x; 1.0132x over previous
import functools

import jax
import jax.numpy as jnp
from jax import lax
from jax.experimental import pallas as pl
from jax.experimental.pallas import tpu as pltpu
from jax.experimental.pallas import tpu_sc as plsc

N = 10000
D = 128
NC = 2
NS = 16
CHUNK = 128
LANES = 16


_BLK = 1000


def _mlp_body(h, wbs):
    for w, b in wbs:
        h = jnp.maximum(jnp.dot(h, w[...], preferred_element_type=jnp.float32)
                        + b[...], 0.0)
    return h


def _mlp3_tc(x, W0, b0, W1, b1, W2, b2):
    grid = (N // _BLK,)
    wspec = pl.BlockSpec((D, D), lambda i: (0, 0))
    bspec = pl.BlockSpec((1, D), lambda i: (0, 0))

    def body(x_ref, w0, b0r, w1, b1r, w2, b2r, o_ref):
        o_ref[...] = _mlp_body(x_ref[...], ((w0, b0r), (w1, b1r), (w2, b2r)))

    return pl.pallas_call(
        body,
        grid=grid,
        in_specs=[pl.BlockSpec((_BLK, D), lambda i: (i, 0)),
                  wspec, bspec, wspec, bspec, wspec, bspec],
        out_specs=pl.BlockSpec((_BLK, D), lambda i: (i, 0)),
        out_shape=jax.ShapeDtypeStruct((N, D), jnp.float32),
    )(x, W0, b0.reshape(1, D), W1, b1.reshape(1, D), W2, b2.reshape(1, D))


def _combine_mlp3_tc(parts, W0, b0, W1, b1, W2, b2):
    grid = (N // _BLK,)
    wspec = pl.BlockSpec((D, D), lambda i: (0, 0))
    bspec = pl.BlockSpec((1, D), lambda i: (0, 0))

    def body(p0_ref, p1_ref, w0, b0r, w1, b1r, w2, b2r, s_ref, h_ref):
        x = p0_ref[0] + p1_ref[0]
        s_ref[...] = x
        h_ref[...] = _mlp_body(x, ((w0, b0r), (w1, b1r), (w2, b2r)))

    return pl.pallas_call(
        body,
        grid=grid,
        in_specs=[pl.BlockSpec((1, _BLK, D), lambda i: (0, i, 0)),
                  pl.BlockSpec((1, _BLK, D), lambda i: (1, i, 0)),
                  wspec, bspec, wspec, bspec, wspec, bspec],
        out_specs=[pl.BlockSpec((_BLK, D), lambda i: (i, 0)),
                   pl.BlockSpec((_BLK, D), lambda i: (i, 0))],
        out_shape=[jax.ShapeDtypeStruct((N, D), jnp.float32),
                   jax.ShapeDtypeStruct((N, D), jnp.float32)],
    )(parts, parts, W0, b0.reshape(1, D), W1, b1.reshape(1, D),
      W2, b2.reshape(1, D))


def _combine_tc(parts):
    grid = (N // _BLK,)

    def body(p0_ref, p1_ref, o_ref):
        o_ref[...] = p0_ref[0] + p1_ref[0]

    return pl.pallas_call(
        body,
        grid=grid,
        in_specs=[pl.BlockSpec((1, _BLK, D), lambda i: (0, i, 0)),
                  pl.BlockSpec((1, _BLK, D), lambda i: (1, i, 0))],
        out_specs=pl.BlockSpec((_BLK, D), lambda i: (i, 0)),
        out_shape=jax.ShapeDtypeStruct((N, D), jnp.float32),
    )(parts, parts)


def _prep_edges(indices, values):
    e = values.shape[0]
    per_tile = -(-e // (NC * NS * CHUNK)) * CHUNK
    epad = per_tile * NC * NS
    pad = epad - e
    row = jnp.concatenate([indices[0], jnp.zeros((pad,), jnp.int32)])
    col = jnp.concatenate([indices[1], jnp.zeros((pad,), jnp.int32)])
    val = jnp.concatenate([values, jnp.zeros((pad,), jnp.float32)])
    shape = (NC, NS, per_tile // CHUNK, CHUNK)
    return row.reshape(shape), col.reshape(shape), val.reshape(shape)


def _spmm_sc(h, row, col, val):
    nchunks = row.shape[2]
    rpt = (N // NS) // 8 * 8
    tail = N - NS * rpt
    nz_full = rpt // CHUNK
    nz_rem = rpt % CHUNK
    mesh = plsc.VectorSubcoreMesh(core_axis_name="c", subcore_axis_name="s")

    @functools.partial(
        pl.kernel,
        out_type=jax.ShapeDtypeStruct((NC, N, D), jnp.float32),
        mesh=mesh,
        scratch_types=[
            pltpu.VMEM((nchunks, CHUNK), jnp.int32),
            pltpu.VMEM((nchunks, CHUNK), jnp.int32),
            pltpu.VMEM((nchunks, CHUNK), jnp.float32),
            pltpu.VMEM((CHUNK, D), jnp.float32),
            pltpu.VMEM_SHARED((N, D), jnp.float32),
            pltpu.SemaphoreType.DMA,
            pltpu.SemaphoreType.DMA,
        ],
    )
    def k(h_hbm, row_hbm, col_hbm, val_hbm, out_hbm,
          row_v, col_v, val_v, gbuf, acc, gsem, gsem2):
        c = lax.axis_index("c")
        s = lax.axis_index("s")

        pltpu.sync_copy(row_hbm.at[c, s], row_v)
        pltpu.sync_copy(col_hbm.at[c, s], col_v)
        pltpu.sync_copy(val_hbm.at[c, s], val_v)

        zero = jnp.zeros((LANES,), jnp.float32)

        def zbody(r, _):
            for q in range(D // LANES):
                gbuf[r, pl.ds(q * LANES, LANES)] = zero
            return 0

        lax.fori_loop(0, CHUNK, zbody, 0)
        base = s * rpt
        for t in range(nz_full):
            pltpu.sync_copy(gbuf, acc.at[pl.ds(base + t * CHUNK, CHUNK)])
        if nz_rem:
            pltpu.sync_copy(gbuf.at[pl.ds(0, nz_rem)],
                            acc.at[pl.ds(base + nz_full * CHUNK, nz_rem)])
        if tail:
            @pl.when(s == NS - 1)
            def _():
                pltpu.sync_copy(gbuf.at[pl.ds(0, tail)],
                                acc.at[pl.ds(NS * rpt, tail)])
        plsc.subcore_barrier()

        half = CHUNK // 2
        dnums = lax.GatherDimensionNumbers(
            offset_dims=(), collapsed_slice_dims=(0,), start_index_map=(0,))

        def scale_half(j, lo):
            def gbody(g, _):
                vv = val_v[j, pl.ds(lo + g * LANES, LANES)]
                for i in range(LANES):
                    b = lax.gather(
                        vv, jnp.full((LANES, 1), i, jnp.int32), dnums,
                        slice_sizes=(1,),
                        mode=lax.GatherScatterMode.PROMISE_IN_BOUNDS)
                    e = lo + g * LANES + i
                    for q in range(D // LANES):
                        sl = pl.ds(q * LANES, LANES)
                        gbuf[e, sl] = gbuf[e, sl] * b
                return 0

            lax.fori_loop(0, half // LANES, gbody, 0)

        def chunk_body(j, _):
            ga = pltpu.async_copy(h_hbm.at[col_v.at[j, pl.ds(0, half)]],
                                  gbuf.at[pl.ds(0, half)], gsem)
            gb = pltpu.async_copy(h_hbm.at[col_v.at[j, pl.ds(half, half)]],
                                  gbuf.at[pl.ds(half, half)], gsem2)
            ga.wait()
            scale_half(j, 0)
            gb.wait()
            scale_half(j, half)
            pltpu.sync_copy(gbuf, acc.at[row_v.at[j]], add=True)
            return 0

        lax.fori_loop(0, nchunks, chunk_body, 0)
        plsc.subcore_barrier()

        pltpu.sync_copy(acc.at[pl.ds(base, rpt)],
                        out_hbm.at[c, pl.ds(base, rpt)])
        if tail:
            @pl.when(s == NS - 1)
            def _():
                pltpu.sync_copy(acc.at[pl.ds(NS * rpt, tail)],
                                out_hbm.at[c, pl.ds(NS * rpt, tail)])

    return k(h, row, col, val)


def kernel(inputs, summ0_indices, summ0_values, summ1_indices, summ1_values,
           dag_W0, dag_b0, dag_W1, dag_b1, dag_W2, dag_b2,
           glob_W0, glob_b0, glob_W1, glob_b1, glob_W2, glob_b2):
    h0 = _mlp3_tc(inputs, dag_W0, dag_b0, dag_W1, dag_b1, dag_W2, dag_b2)
    r0, c0, v0 = _prep_edges(summ0_indices, summ0_values)
    parts0 = _spmm_sc(h0, r0, c0, v0)
    out0, h1 = _combine_mlp3_tc(parts0, glob_W0, glob_b0, glob_W1, glob_b1,
                                glob_W2, glob_b2)
    r1, c1, v1 = _prep_edges(summ1_indices, summ1_values)
    parts1 = _spmm_sc(h1, r1, c1, v1)
    out1 = _combine_tc(parts1)
    return (out0, out1)
